# baseline (speedup 1.0000x reference)
import jax
import jax.numpy as jnp
from jax.experimental import pallas as pl
from jax.experimental.pallas import tpu as pltpu


def _dma_copy_kernel(u_ref, i_ref, uo_ref, io_ref, sem_u, sem_i):
    cu = pltpu.make_async_copy(u_ref, uo_ref, sem_u)
    ci = pltpu.make_async_copy(i_ref, io_ref, sem_i)
    cu.start()
    ci.start()
    cu.wait()
    ci.wait()


def kernel(user_w, item_w, edge_index):
    del edge_index
    u2 = user_w.reshape(25000, 256)
    i2 = item_w.reshape(25000, 2560)
    user_out, item_out = pl.pallas_call(
        _dma_copy_kernel,
        in_specs=[
            pl.BlockSpec(memory_space=pl.ANY),
            pl.BlockSpec(memory_space=pl.ANY),
        ],
        out_specs=[
            pl.BlockSpec(memory_space=pl.ANY),
            pl.BlockSpec(memory_space=pl.ANY),
        ],
        out_shape=[
            jax.ShapeDtypeStruct(u2.shape, u2.dtype),
            jax.ShapeDtypeStruct(i2.shape, i2.dtype),
        ],
        scratch_shapes=[pltpu.SemaphoreType.DMA, pltpu.SemaphoreType.DMA],
    )(u2, i2)
    return (user_out.reshape(user_w.shape), item_out.reshape(item_w.shape))


# speedup vs baseline: 8.9127x; 8.9127x over previous
import jax
import jax.numpy as jnp
from jax.experimental import pallas as pl
from jax.experimental.pallas import tpu as pltpu

_R = 10000
_B = 8
_L = 4


def _memcpy_kernel(u_ref, i_ref, uo_ref, io_ref, *scratch):
    bufs = scratch[:_B]
    sem_in = scratch[_B:2 * _B]
    sem_out = scratch[2 * _B:3 * _B]

    chunks = []
    for c in range(i_ref.shape[0] // _R):
        sl = pl.ds(c * _R, _R)
        chunks.append((i_ref.at[sl], io_ref.at[sl]))
    for c in range(u_ref.shape[0] // _R):
        sl = pl.ds(c * _R, _R)
        chunks.append((u_ref.at[sl], uo_ref.at[sl]))
    n = len(chunks)

    def in_copy(k):
        return pltpu.make_async_copy(chunks[k][0], bufs[k % _B], sem_in[k % _B])

    def out_copy(k):
        return pltpu.make_async_copy(bufs[k % _B], chunks[k][1], sem_out[k % _B])

    for k in range(min(_L, n)):
        in_copy(k).start()
    for k in range(n):
        in_copy(k).wait()
        out_copy(k).start()
        nk = k + _L
        if nk < n:
            if nk - _B >= 0:
                out_copy(nk - _B).wait()
            in_copy(nk).start()
    for k in range(max(0, n - _B), n):
        out_copy(k).wait()


def kernel(user_w, item_w, edge_index):
    del edge_index
    user_out, item_out = pl.pallas_call(
        _memcpy_kernel,
        in_specs=[
            pl.BlockSpec(memory_space=pl.ANY),
            pl.BlockSpec(memory_space=pl.ANY),
        ],
        out_specs=[
            pl.BlockSpec(memory_space=pl.ANY),
            pl.BlockSpec(memory_space=pl.ANY),
        ],
        out_shape=[
            jax.ShapeDtypeStruct(user_w.shape, user_w.dtype),
            jax.ShapeDtypeStruct(item_w.shape, item_w.dtype),
        ],
        scratch_shapes=(
            [pltpu.VMEM((_R, 64), jnp.float32)] * _B
            + [pltpu.SemaphoreType.DMA] * (2 * _B)
        ),
    )(user_w, item_w)
    return (user_out, item_out)
